# baseline (speedup 1.0000x reference)
import functools

import jax
import jax.numpy as jnp
from jax import lax
from jax.experimental import pallas as pl
from jax.experimental.pallas import tpu as pltpu
from jax.experimental.pallas import tpu_sc as plsc

N_EMB = 8192
DIM = 256
N_TOK = 4096
COMMITMENT_COST = 0.25
TM = 512


def _dist_body(x_ref, w_ref, sx_ref, sw_ref, idx_ref, cnt_ref, enc_ref):
    i = pl.program_id(0)
    x = x_ref[...]
    w = w_ref[...]
    s2 = lax.dot_general(x * (-2.0), w, (((1,), (1,)), ((), ())),
                         preferred_element_type=jnp.float32)
    sx = sx_ref[...]
    LW = 128
    RB = 64
    ik0r = lax.broadcasted_iota(jnp.int32, (RB, LW), 1)
    ik0 = lax.broadcasted_iota(jnp.int32, (TM, LW), 1)
    idx_parts = []
    for r in range(TM // RB):
        rs = slice(r * RB, (r + 1) * RB)
        sxr = sx[rs]
        m = mi = None
        for k in range(N_EMB // LW):
            ks = slice(k * LW, (k + 1) * LW)
            dk = (sxr + sw_ref[:, ks]) + s2[rs, ks]
            if k == 0:
                m, mi = dk, jnp.zeros((RB, LW), jnp.int32)
            else:
                lt = dk < m
                m = jnp.where(lt, dk, m)
                mi = jnp.where(lt, jnp.full((RB, LW), k, jnp.int32), mi)
        mv = jnp.min(m, axis=1, keepdims=True)
        idx_parts.append(jnp.min(
            jnp.where(m == mv, mi * LW + ik0r, N_EMB), axis=1, keepdims=True))
    idx = jnp.concatenate(idx_parts, axis=0)
    idx_ref[...] = idx
    cs = []
    for k in range(N_EMB // LW):
        ohk = jnp.where(ik0 + (k * LW) == idx, 1.0, 0.0).astype(jnp.float32)
        enc_ref[:, k * LW:(k + 1) * LW] = ohk
        cs.append(jnp.sum(ohk, axis=0, keepdims=True))

    @pl.when(i == 0)
    def _():
        cnt_ref[...] = jnp.zeros_like(cnt_ref)

    cnt_ref[...] += jnp.concatenate(cs, axis=1)


def _tc_distance(x, w, sx, sw):
    return pl.pallas_call(
        _dist_body,
        grid=(N_TOK // TM,),
        in_specs=[
            pl.BlockSpec((TM, DIM), lambda i: (i, 0)),
            pl.BlockSpec((N_EMB, DIM), lambda i: (0, 0)),
            pl.BlockSpec((TM, 1), lambda i: (i, 0)),
            pl.BlockSpec((1, N_EMB), lambda i: (0, 0)),
        ],
        out_specs=[
            pl.BlockSpec((TM, 1), lambda i: (i, 0)),
            pl.BlockSpec((1, N_EMB), lambda i: (0, 0)),
            pl.BlockSpec((TM, N_EMB), lambda i: (i, 0)),
        ],
        out_shape=[
            jax.ShapeDtypeStruct((N_TOK, 1), jnp.int32),
            jax.ShapeDtypeStruct((1, N_EMB), jnp.float32),
            jax.ShapeDtypeStruct((N_TOK, N_EMB), jnp.float32),
        ],
    )(x, w, sx, sw)


def _sc_gather(w, idx):
    info = plsc.get_sparse_core_info()
    nw = info.num_cores * info.num_subcores
    bpw = N_TOK // nw
    mesh = plsc.VectorSubcoreMesh(core_axis_name="c", subcore_axis_name="s")

    @functools.partial(
        pl.kernel,
        mesh=mesh,
        out_type=jax.ShapeDtypeStruct((N_TOK, DIM), jnp.float32),
        scratch_types=[
            pltpu.VMEM((bpw,), jnp.int32),
            pltpu.VMEM((bpw, DIM), jnp.float32),
            pltpu.SemaphoreType.DMA,
        ],
    )
    def k(w_hbm, idx_hbm, out_hbm, idx_v, rows_v, sem):
        wid = lax.axis_index("s") * info.num_cores + lax.axis_index("c")
        base = wid * bpw
        pltpu.sync_copy(idx_hbm.at[pl.ds(base, bpw)], idx_v)
        pltpu.async_copy(w_hbm.at[idx_v], rows_v, sem).wait()
        pltpu.sync_copy(rows_v, out_hbm.at[pl.ds(base, bpw)])

    return k(w, idx)


TL = 512


def _loss_body(x_ref, q_ref, c_ref, qst_ref, loss_ref, perp_ref):
    i = pl.program_id(0)
    x = x_ref[...]
    q = q_ref[...]
    d = q - x
    qst_ref[...] = x + d
    part = jnp.sum(d * d)

    @pl.when(i == 0)
    def _():
        loss_ref[0, 0] = 0.0

    loss_ref[0, 0] += part

    @pl.when(i == pl.num_programs(0) - 1)
    def _():
        m = loss_ref[0, 0] * (1.0 / (N_TOK * DIM))
        loss_ref[0, 0] = m + COMMITMENT_COST * m
        p = c_ref[...] * (1.0 / N_TOK)
        perp_ref[0, 0] = jnp.exp(-jnp.sum(p * jnp.log(p + 1e-10)))


def _tc_loss(x, q, counts):
    return pl.pallas_call(
        _loss_body,
        grid=(N_TOK // TL,),
        in_specs=[
            pl.BlockSpec((TL, DIM), lambda i: (i, 0)),
            pl.BlockSpec((TL, DIM), lambda i: (i, 0)),
            pl.BlockSpec((1, N_EMB), lambda i: (0, 0)),
        ],
        out_specs=[
            pl.BlockSpec((TL, DIM), lambda i: (i, 0)),
            pl.BlockSpec(memory_space=pltpu.SMEM),
            pl.BlockSpec(memory_space=pltpu.SMEM),
        ],
        out_shape=[
            jax.ShapeDtypeStruct((N_TOK, DIM), jnp.float32),
            jax.ShapeDtypeStruct((1, 1), jnp.float32),
            jax.ShapeDtypeStruct((1, 1), jnp.float32),
        ],
    )(x, q, counts)


def kernel(inputs, weight):
    x = jnp.transpose(inputs, (0, 2, 3, 4, 1)).reshape(N_TOK, DIM)
    sx = jnp.sum(x ** 2, axis=1, keepdims=True)
    sw = jnp.sum(weight ** 2, axis=1).reshape(1, N_EMB)
    idx2, counts, enc = _tc_distance(x, weight, sx, sw)
    q = _sc_gather(weight, idx2.reshape(N_TOK))
    qst, loss, perp = _tc_loss(x, q, counts)
    quantized_out = jnp.transpose(qst.reshape(4, 4, 16, 16, DIM), (0, 4, 1, 2, 3))
    return (loss.reshape(()), quantized_out, perp.reshape(()), enc)


# speedup vs baseline: 1.0281x; 1.0281x over previous
import functools

import jax
import jax.numpy as jnp
from jax import lax
from jax.experimental import pallas as pl
from jax.experimental.pallas import tpu as pltpu
from jax.experimental.pallas import tpu_sc as plsc

N_EMB = 8192
DIM = 256
N_TOK = 4096
COMMITMENT_COST = 0.25
TM = 512


def _dist_body(x_ref, w_ref, sx_ref, sw_ref, idx_ref, cnt_ref, enc_ref):
    i = pl.program_id(0)
    x = x_ref[...]
    w = w_ref[...]
    s2 = lax.dot_general(x * (-2.0), w, (((1,), (1,)), ((), ())),
                         preferred_element_type=jnp.float32)
    sx = sx_ref[...]
    LW = 128
    RB = 64
    ik0r = lax.broadcasted_iota(jnp.int32, (RB, LW), 1)
    ik0 = lax.broadcasted_iota(jnp.int32, (TM, LW), 1)
    idx_parts = []
    for r in range(TM // RB):
        rs = slice(r * RB, (r + 1) * RB)
        sxr = sx[rs]
        m = mi = None
        for k in range(N_EMB // LW):
            ks = slice(k * LW, (k + 1) * LW)
            dk = (sxr + sw_ref[:, ks]) + s2[rs, ks]
            if k == 0:
                m, mi = dk, jnp.zeros((RB, LW), jnp.int32)
            else:
                lt = dk < m
                m = jnp.where(lt, dk, m)
                mi = jnp.where(lt, jnp.full((RB, LW), k, jnp.int32), mi)
        mv = jnp.min(m, axis=1, keepdims=True)
        idx_parts.append(jnp.min(
            jnp.where(m == mv, mi * LW + ik0r, N_EMB), axis=1, keepdims=True))
    idx = jnp.concatenate(idx_parts, axis=0)
    idx_ref[...] = idx
    cs = []
    for k in range(N_EMB // LW):
        ohk = jnp.where(ik0 + (k * LW) == idx, 1.0, 0.0).astype(jnp.float32)
        enc_ref[:, k * LW:(k + 1) * LW] = ohk
        cs.append(jnp.sum(ohk, axis=0, keepdims=True))

    @pl.when(i == 0)
    def _():
        cnt_ref[...] = jnp.zeros_like(cnt_ref)

    cnt_ref[...] += jnp.concatenate(cs, axis=1)


def _tc_distance(x, w, sx, sw):
    return pl.pallas_call(
        _dist_body,
        grid=(N_TOK // TM,),
        in_specs=[
            pl.BlockSpec((TM, DIM), lambda i: (i, 0)),
            pl.BlockSpec((N_EMB, DIM), lambda i: (0, 0)),
            pl.BlockSpec((TM, 1), lambda i: (i, 0)),
            pl.BlockSpec((1, N_EMB), lambda i: (0, 0)),
        ],
        out_specs=[
            pl.BlockSpec((TM, 1), lambda i: (i, 0)),
            pl.BlockSpec((1, N_EMB), lambda i: (0, 0)),
            pl.BlockSpec((TM, N_EMB), lambda i: (i, 0)),
        ],
        out_shape=[
            jax.ShapeDtypeStruct((N_TOK, 1), jnp.int32),
            jax.ShapeDtypeStruct((1, N_EMB), jnp.float32),
            jax.ShapeDtypeStruct((N_TOK, N_EMB), jnp.float32),
        ],
    )(x, w, sx, sw)


def _sc_gather(w, idx):
    info = plsc.get_sparse_core_info()
    nw = info.num_cores * info.num_subcores
    bpw = N_TOK // nw
    mesh = plsc.VectorSubcoreMesh(core_axis_name="c", subcore_axis_name="s")

    @functools.partial(
        pl.kernel,
        mesh=mesh,
        out_type=jax.ShapeDtypeStruct((N_TOK, DIM), jnp.float32),
        scratch_types=[
            pltpu.VMEM((bpw,), jnp.int32),
            pltpu.VMEM((bpw, DIM), jnp.float32),
            pltpu.SemaphoreType.DMA,
        ],
    )
    def k(w_hbm, idx_hbm, out_hbm, idx_v, rows_v, sem):
        wid = lax.axis_index("s") * info.num_cores + lax.axis_index("c")
        base = wid * bpw
        pltpu.sync_copy(idx_hbm.at[pl.ds(base, bpw)], idx_v)
        pltpu.async_copy(w_hbm.at[idx_v], rows_v, sem).wait()
        pltpu.sync_copy(rows_v, out_hbm.at[pl.ds(base, bpw)])

    return k(w, idx)


def _loss_body(x_ref, q_ref, c_ref, qst_ref, loss_ref, perp_ref):
    x = x_ref[...]
    q = q_ref[...]
    d = q - x
    qst_ref[...] = x + d
    m = jnp.sum(d * d) * (1.0 / (N_TOK * DIM))
    loss_ref[0, 0] = m + COMMITMENT_COST * m
    p = c_ref[...] * (1.0 / N_TOK)
    perp_ref[0, 0] = jnp.exp(-jnp.sum(p * jnp.log(p + 1e-10)))


def _tc_loss(x, q, counts):
    return pl.pallas_call(
        _loss_body,
        in_specs=[
            pl.BlockSpec(memory_space=pltpu.VMEM),
            pl.BlockSpec(memory_space=pltpu.VMEM),
            pl.BlockSpec(memory_space=pltpu.VMEM),
        ],
        out_specs=[
            pl.BlockSpec(memory_space=pltpu.VMEM),
            pl.BlockSpec(memory_space=pltpu.SMEM),
            pl.BlockSpec(memory_space=pltpu.SMEM),
        ],
        out_shape=[
            jax.ShapeDtypeStruct((N_TOK, DIM), jnp.float32),
            jax.ShapeDtypeStruct((1, 1), jnp.float32),
            jax.ShapeDtypeStruct((1, 1), jnp.float32),
        ],
    )(x, q, counts)


def kernel(inputs, weight):
    x = jnp.transpose(inputs, (0, 2, 3, 4, 1)).reshape(N_TOK, DIM)
    sx = jnp.sum(x ** 2, axis=1, keepdims=True)
    sw = jnp.sum(weight ** 2, axis=1).reshape(1, N_EMB)
    idx2, counts, enc = _tc_distance(x, weight, sx, sw)
    q = _sc_gather(weight, idx2.reshape(N_TOK))
    qst, loss, perp = _tc_loss(x, q, counts)
    quantized_out = jnp.transpose(qst.reshape(4, 4, 16, 16, DIM), (0, 4, 1, 2, 3))
    return (loss.reshape(()), quantized_out, perp.reshape(()), enc)
